# baseline (speedup 1.0000x reference)
import functools

import jax
import jax.numpy as jnp
import numpy as np
from jax import lax
from jax.experimental import pallas as pl
from jax.experimental.pallas import tpu as pltpu
from jax.experimental.pallas import tpu_sc as plsc

NUM_LEVELS = 16
LEVEL_DIM = 2
BASE_RES = 16
LOG2_T = 19
T = 1 << LOG2_T
SCALE = 2.0

PRIME_Y = np.uint32(2654435761).astype(np.int32)
PRIME_Z = np.int32(805459861)

_OFFS = [(i, j, k) for i in (0, 1) for j in (0, 1) for k in (0, 1)]

NC = 2
NS = 16
NW = NC * NS
L = 16

B = 256
GRP = B // L


def _iota():
    return lax.iota(jnp.int32, L)


def _level_res(lvl):
    return int(np.floor(BASE_RES * (SCALE ** lvl)))


def _corner_indices(x0, y0, z0, lvl):
    res = _level_res(lvl)
    stride = res + 1
    lbase = lvl * T
    if stride ** 3 <= T:
        base_i = x0 + y0 * stride + z0 * (stride * stride) + lbase
        return [
            base_i + (dx + dy * stride + dz * stride * stride)
            for (dx, dy, dz) in _OFFS
        ]
    hx = (x0, x0 + 1)
    hy0 = y0 * PRIME_Y
    hy = (hy0, hy0 + PRIME_Y)
    hz0 = z0 * PRIME_Z
    hz = (hz0, hz0 + PRIME_Z)
    return [
        ((hx[dx] ^ hy[dy] ^ hz[dz]) & (T - 1)) + lbase
        for (dx, dy, dz) in _OFFS
    ]


@functools.lru_cache(maxsize=None)
def _build(n_points):
    ppw = n_points // NW
    nb = ppw // B

    mesh = plsc.VectorSubcoreMesh(core_axis_name="c", subcore_axis_name="s")

    @functools.partial(
        pl.kernel,
        mesh=mesh,
        out_type=jax.ShapeDtypeStruct((n_points, NUM_LEVELS * LEVEL_DIM),
                                      jnp.float32),
        compiler_params=pltpu.CompilerParams(
            needs_layout_passes=False,
            use_tc_tiling_on_sc=False,
        ),
        scratch_types=[
            pltpu.VMEM((B,), jnp.float32),
            pltpu.VMEM((B,), jnp.float32),
            pltpu.VMEM((B,), jnp.float32),
            pltpu.VMEM((GRP, 8 * L), jnp.int32),
            pltpu.VMEM((GRP, 8 * L, LEVEL_DIM), jnp.float32),
            pltpu.VMEM((B, NUM_LEVELS * LEVEL_DIM), jnp.float32),
            pltpu.VMEM((3,), jnp.float32),
            pltpu.SemaphoreType.DMA,
        ],
    )
    def grid_kernel(x_hbm, aabb_hbm, tab_hbm, out_hbm,
                    xs, ys, zs, ib, vb, outb, abuf, sem_g):
        wid = lax.axis_index("s") * NC + lax.axis_index("c")
        base0 = wid * ppw
        pltpu.sync_copy(aabb_hbm, abuf)

        def pos_frac(g, lvl):
            sl = pl.ds(g * L, L)
            res = float(_level_res(lvl))
            px, py, pz = xs[sl] * res, ys[sl] * res, zs[sl] * res
            x0 = px.astype(jnp.int32)
            y0 = py.astype(jnp.int32)
            z0 = pz.astype(jnp.int32)
            fx = px - x0.astype(jnp.float32)
            fy = py - y0.astype(jnp.float32)
            fz = pz - z0.astype(jnp.float32)
            return x0, y0, z0, fx, fy, fz

        def phase1(lvl):
            def p1(g, c0):
                x0, y0, z0, _, _, _ = pos_frac(g, lvl)
                corners = _corner_indices(x0, y0, z0, lvl)
                for c in range(8):
                    ib[g, pl.ds(c * L, L)] = corners[c]
                return c0

            lax.fori_loop(0, GRP, p1, 0)

        def fire(lvl):
            return [
                pltpu.async_copy(tab_hbm.at[ib.at[g]], vb.at[g], sem_g)
                for g in range(GRP)
            ]

        def phase3(lvl):
            def p3(g, c0):
                _, _, _, fx, fy, fz = pos_frac(g, lvl)
                wx = (1.0 - fx, fx)
                wy = (1.0 - fy, fy)
                wz = (1.0 - fz, fz)
                wxy = [wx[i] * wy[j] for i in (0, 1) for j in (0, 1)]
                acc0 = jnp.zeros((L,), jnp.float32)
                acc1 = jnp.zeros((L,), jnp.float32)
                gv = jnp.full((L,), g, jnp.int32)
                for c, (dx, dy, dz) in enumerate(_OFFS):
                    w = wxy[dx * 2 + dy] * wz[dz]
                    rows = c * L + _iota()
                    f0 = plsc.load_gather(
                        vb, [gv, rows, jnp.zeros((L,), jnp.int32)])
                    f1 = plsc.load_gather(
                        vb, [gv, rows, jnp.ones((L,), jnp.int32)])
                    acc0 = acc0 + w * f0
                    acc1 = acc1 + w * f1
                pts = g * L + _iota()
                plsc.store_scatter(
                    outb, [pts, jnp.full((L,), 2 * lvl, jnp.int32)], acc0)
                plsc.store_scatter(
                    outb, [pts, jnp.full((L,), 2 * lvl + 1, jnp.int32)], acc1)
                return c0

            lax.fori_loop(0, GRP, p3, 0)

        def block(b, carry):
            base = base0 + b * B
            for c, buf in ((0, xs), (1, ys), (2, zs)):
                pltpu.sync_copy(x_hbm.at[pl.ds(c * n_points + base, B)], buf)

            def p0(g, c0):
                sl = pl.ds(g * L, L)
                for c, buf in ((0, xs), (1, ys), (2, zs)):
                    a = plsc.load_gather(abuf, [jnp.full((L,), c, jnp.int32)])
                    buf[sl] = (buf[sl] / a + 1.0) * 0.5
                return c0

            lax.fori_loop(0, GRP, p0, 0)

            for lvl in range(NUM_LEVELS):
                phase1(lvl)
                cps = fire(lvl)
                for cp in cps:
                    cp.wait()
                phase3(lvl)

            pltpu.sync_copy(outb, out_hbm.at[pl.ds(base, B)])
            return carry

        lax.fori_loop(0, nb, block, 0)

    return grid_kernel


def kernel(x, aabb, tables):
    n = x.shape[0]
    x_flat = jnp.transpose(x).reshape(-1)
    tab2d = tables.reshape(NUM_LEVELS * T, LEVEL_DIM)
    return _build(n)(x_flat, aabb, tab2d)


# speedup vs baseline: 1.0975x; 1.0975x over previous
import functools

import jax
import jax.numpy as jnp
import numpy as np
from jax import lax
from jax.experimental import pallas as pl
from jax.experimental.pallas import tpu as pltpu
from jax.experimental.pallas import tpu_sc as plsc

NUM_LEVELS = 16
LEVEL_DIM = 2
BASE_RES = 16
LOG2_T = 19
T = 1 << LOG2_T
SCALE = 2.0

PRIME_Y = np.uint32(2654435761).astype(np.int32)
PRIME_Z = np.int32(805459861)

_OFFS = [(i, j, k) for i in (0, 1) for j in (0, 1) for k in (0, 1)]

NC = 2
NS = 16
NW = NC * NS
L = 16

B = 256
GRP = B // L


def _iota():
    return lax.iota(jnp.int32, L)


def _level_res(lvl):
    return int(np.floor(BASE_RES * (SCALE ** lvl)))


def _corner_indices(x0, y0, z0, lvl):
    res = _level_res(lvl)
    stride = res + 1
    lbase = lvl * T
    if stride ** 3 <= T:
        base_i = x0 + y0 * stride + z0 * (stride * stride) + lbase
        return [
            base_i + (dx + dy * stride + dz * stride * stride)
            for (dx, dy, dz) in _OFFS
        ]
    hx = (x0, x0 + 1)
    hy0 = y0 * PRIME_Y
    hy = (hy0, hy0 + PRIME_Y)
    hz0 = z0 * PRIME_Z
    hz = (hz0, hz0 + PRIME_Z)
    return [
        ((hx[dx] ^ hy[dy] ^ hz[dz]) & (T - 1)) + lbase
        for (dx, dy, dz) in _OFFS
    ]


@functools.lru_cache(maxsize=None)
def _build(n_points):
    ppw = n_points // NW
    nb = ppw // B

    mesh = plsc.VectorSubcoreMesh(core_axis_name="c", subcore_axis_name="s")

    @functools.partial(
        pl.kernel,
        mesh=mesh,
        out_type=jax.ShapeDtypeStruct((n_points, NUM_LEVELS * LEVEL_DIM),
                                      jnp.float32),
        compiler_params=pltpu.CompilerParams(
            needs_layout_passes=False,
            use_tc_tiling_on_sc=False,
        ),
        scratch_types=[
            pltpu.VMEM((B,), jnp.float32),
            pltpu.VMEM((B,), jnp.float32),
            pltpu.VMEM((B,), jnp.float32),
            pltpu.VMEM((GRP, 8 * L), jnp.int32),
            pltpu.VMEM((GRP, 8 * L), jnp.int32),
            pltpu.VMEM((GRP, 8 * L, LEVEL_DIM), jnp.float32),
            pltpu.VMEM((GRP, 8 * L, LEVEL_DIM), jnp.float32),
            pltpu.VMEM((B, NUM_LEVELS * LEVEL_DIM), jnp.float32),
            pltpu.VMEM((3,), jnp.float32),
            pltpu.SemaphoreType.DMA,
            pltpu.SemaphoreType.DMA,
        ],
    )
    def grid_kernel(x_hbm, aabb_hbm, tab_hbm, out_hbm,
                    xs, ys, zs, ib_e, ib_o, vb_e, vb_o, outb, abuf,
                    sem_e, sem_o):
        parity = [(ib_e, vb_e, sem_e), (ib_o, vb_o, sem_o)]
        wid = lax.axis_index("s") * NC + lax.axis_index("c")
        base0 = wid * ppw
        pltpu.sync_copy(aabb_hbm, abuf)

        def pos_frac(g, lvl):
            sl = pl.ds(g * L, L)
            res = float(_level_res(lvl))
            px, py, pz = xs[sl] * res, ys[sl] * res, zs[sl] * res
            x0 = px.astype(jnp.int32)
            y0 = py.astype(jnp.int32)
            z0 = pz.astype(jnp.int32)
            fx = px - x0.astype(jnp.float32)
            fy = py - y0.astype(jnp.float32)
            fz = pz - z0.astype(jnp.float32)
            return x0, y0, z0, fx, fy, fz

        def phase1(lvl):
            ib = parity[lvl % 2][0]

            def p1(g, c0):
                x0, y0, z0, _, _, _ = pos_frac(g, lvl)
                corners = _corner_indices(x0, y0, z0, lvl)
                for c in range(8):
                    ib[g, pl.ds(c * L, L)] = corners[c]
                return c0

            lax.fori_loop(0, GRP, p1, 0)

        def fire(lvl):
            ib, vb, sem = parity[lvl % 2]
            return [
                pltpu.async_copy(tab_hbm.at[ib.at[g]], vb.at[g], sem)
                for g in range(GRP)
            ]

        def phase3(lvl):
            vb = parity[lvl % 2][1]
            def p3(g, c0):
                _, _, _, fx, fy, fz = pos_frac(g, lvl)
                wx = (1.0 - fx, fx)
                wy = (1.0 - fy, fy)
                wz = (1.0 - fz, fz)
                wxy = [wx[i] * wy[j] for i in (0, 1) for j in (0, 1)]
                acc0 = jnp.zeros((L,), jnp.float32)
                acc1 = jnp.zeros((L,), jnp.float32)
                gv = jnp.full((L,), g, jnp.int32)
                for c, (dx, dy, dz) in enumerate(_OFFS):
                    w = wxy[dx * 2 + dy] * wz[dz]
                    rows = c * L + _iota()
                    f0 = plsc.load_gather(
                        vb, [gv, rows, jnp.zeros((L,), jnp.int32)])
                    f1 = plsc.load_gather(
                        vb, [gv, rows, jnp.ones((L,), jnp.int32)])
                    acc0 = acc0 + w * f0
                    acc1 = acc1 + w * f1
                pts = g * L + _iota()
                plsc.store_scatter(
                    outb, [pts, jnp.full((L,), 2 * lvl, jnp.int32)], acc0)
                plsc.store_scatter(
                    outb, [pts, jnp.full((L,), 2 * lvl + 1, jnp.int32)], acc1)
                return c0

            lax.fori_loop(0, GRP, p3, 0)

        def block(b, carry):
            base = base0 + b * B
            for c, buf in ((0, xs), (1, ys), (2, zs)):
                pltpu.sync_copy(x_hbm.at[pl.ds(c * n_points + base, B)], buf)

            def p0(g, c0):
                sl = pl.ds(g * L, L)
                for c, buf in ((0, xs), (1, ys), (2, zs)):
                    a = plsc.load_gather(abuf, [jnp.full((L,), c, jnp.int32)])
                    buf[sl] = (buf[sl] / a + 1.0) * 0.5
                return c0

            lax.fori_loop(0, GRP, p0, 0)

            phase1(0)
            pend = fire(0)
            for lvl in range(1, NUM_LEVELS):
                phase1(lvl)
                nxt = fire(lvl)
                for cp in pend:
                    cp.wait()
                phase3(lvl - 1)
                pend = nxt
            for cp in pend:
                cp.wait()
            phase3(NUM_LEVELS - 1)

            pltpu.sync_copy(outb, out_hbm.at[pl.ds(base, B)])
            return carry

        lax.fori_loop(0, nb, block, 0)

    return grid_kernel


def kernel(x, aabb, tables):
    n = x.shape[0]
    x_flat = jnp.transpose(x).reshape(-1)
    tab2d = tables.reshape(NUM_LEVELS * T, LEVEL_DIM)
    return _build(n)(x_flat, aabb, tab2d)
